# baseline (speedup 1.0000x reference)
import functools

import jax
import jax.numpy as jnp
from jax import lax
from jax.experimental import pallas as pl
from jax.experimental.pallas import tpu as pltpu
from jax.experimental.pallas import tpu_sc as plsc

BUCKET_SIZE = 32
DIM = 128

_NC = 2
_NS = 16
_NW = _NC * _NS
_L = 16

_CH_B = 4
_CH_R = _CH_B * BUCKET_SIZE


def _sc_bucket_mean(x):
    bh, n, d = x.shape
    buckets = n // BUCKET_SIZE
    n_chunks = n // _CH_R
    bh_per_w = bh // _NW
    mesh = plsc.VectorSubcoreMesh(core_axis_name="c", subcore_axis_name="s")

    @functools.partial(
        pl.kernel, mesh=mesh,
        out_type=jax.ShapeDtypeStruct((bh, buckets, d), jnp.float32),
        scratch_types=[
            pltpu.VMEM((2, _CH_R, d), jnp.float32),
            pltpu.VMEM((buckets, d), jnp.float32),
            pltpu.SemaphoreType.DMA,
            pltpu.SemaphoreType.DMA,
            pltpu.SemaphoreType.DMA,
        ],
    )
    def sck(x_hbm, o_hbm, ring, acc, sem0, sem1, osem):
        wid = lax.axis_index("s") * _NC + lax.axis_index("c")
        sems = (sem0, sem1)
        for bi in range(bh_per_w):
            b = wid * bh_per_w + bi
            pltpu.async_copy(x_hbm.at[b, pl.ds(0, _CH_R)], ring.at[0], sem0)
            pltpu.async_copy(x_hbm.at[b, pl.ds(_CH_R, _CH_R)], ring.at[1],
                             sem1)

            def pair(i, carry, b=b):
                for s in range(2):
                    t = i * 2 + s
                    pltpu.make_async_copy(
                        x_hbm.at[b, pl.ds(0, _CH_R)], ring.at[s],
                        sems[s]).wait()

                    def bucket(bk, c, s=s, t=t):
                        for j in range(d // _L):
                            a = ring[s, bk * BUCKET_SIZE, pl.ds(j * _L, _L)]
                            for r in range(1, BUCKET_SIZE):
                                a = a + ring[s, bk * BUCKET_SIZE + r,
                                             pl.ds(j * _L, _L)]
                            acc[t * _CH_B + bk, pl.ds(j * _L, _L)] = a * (
                                1.0 / BUCKET_SIZE)
                        return c

                    lax.fori_loop(0, _CH_B, bucket, 0, unroll=False)

                    @pl.when(t + 2 < n_chunks)
                    def _(b=b, s=s, t=t):
                        pltpu.async_copy(
                            x_hbm.at[b, pl.ds((t + 2) * _CH_R, _CH_R)],
                            ring.at[s], sems[s])
                return carry

            lax.fori_loop(0, n_chunks // 2, pair, 0, unroll=False)
            pltpu.async_copy(acc, o_hbm.at[b], osem).wait()

    return sck(x)


def _tc_mean_body(q_ref, o_ref):
    n, d = q_ref.shape[1], q_ref.shape[2]
    buckets = n // BUCKET_SIZE
    o_ref[0] = jnp.sum(
        q_ref[0].reshape(buckets, BUCKET_SIZE, d), axis=1) * (
            1.0 / BUCKET_SIZE)


def _tc_mean(q):
    bh, n, d = q.shape
    buckets = n // BUCKET_SIZE
    return pl.pallas_call(
        _tc_mean_body,
        grid=(bh,),
        in_specs=[pl.BlockSpec((1, n, d), lambda i: (i, 0, 0))],
        out_specs=pl.BlockSpec((1, buckets, d), lambda i: (i, 0, 0)),
        out_shape=jax.ShapeDtypeStruct((bh, buckets, d), jnp.float32),
    )(q)


def _tc_attn_body(sq_ref, sk_ref, o_ref):
    r = jax.lax.dot_general(
        sq_ref[0], sk_ref[0], (((1,), (1,)), ((), ())),
        preferred_element_type=jnp.float32) * (DIM ** -0.5)
    m = jnp.max(r, axis=-1, keepdims=True)
    e = jnp.exp(r - m)
    o_ref[0] = e / jnp.sum(e, axis=-1, keepdims=True)


def _tc_attn(sq, sk):
    bh, buckets, d = sq.shape
    return pl.pallas_call(
        _tc_attn_body,
        grid=(bh,),
        in_specs=[
            pl.BlockSpec((1, buckets, d), lambda i: (i, 0, 0)),
            pl.BlockSpec((1, buckets, d), lambda i: (i, 0, 0)),
        ],
        out_specs=pl.BlockSpec((1, buckets, buckets), lambda i: (i, 0, 0)),
        out_shape=jax.ShapeDtypeStruct((bh, buckets, buckets), jnp.float32),
    )(sq, sk)


def kernel(q, k):
    sk = _sc_bucket_mean(k)
    sq = _tc_mean(q)
    return _tc_attn(sq, sk)


# speedup vs baseline: 1.6892x; 1.6892x over previous
import jax
import jax.numpy as jnp
from jax.experimental import pallas as pl

BUCKET_SIZE = 32
DIM = 128


def _body(q_ref, k_ref, o_ref):
    n, d = q_ref.shape[1], q_ref.shape[2]
    buckets = n // BUCKET_SIZE
    qb = q_ref[0].reshape(buckets, BUCKET_SIZE, d)
    kb = k_ref[0].reshape(buckets, BUCKET_SIZE, d)
    sq = jnp.sum(qb, axis=1) * (1.0 / BUCKET_SIZE)
    sk = jnp.sum(kb, axis=1) * (1.0 / BUCKET_SIZE)
    r = jax.lax.dot_general(
        sq, sk, (((1,), (1,)), ((), ())),
        preferred_element_type=jnp.float32) * (DIM ** -0.5)
    m = jnp.max(r, axis=-1, keepdims=True)
    e = jnp.exp(r - m)
    o_ref[0] = e / jnp.sum(e, axis=-1, keepdims=True)


def kernel(q, k):
    bh, n, d = q.shape
    buckets = n // BUCKET_SIZE
    return pl.pallas_call(
        _body,
        grid=(bh,),
        in_specs=[
            pl.BlockSpec((1, n, d), lambda i: (i, 0, 0)),
            pl.BlockSpec((1, n, d), lambda i: (i, 0, 0)),
        ],
        out_specs=pl.BlockSpec((1, buckets, buckets), lambda i: (i, 0, 0)),
        out_shape=jax.ShapeDtypeStruct((bh, buckets, buckets), jnp.float32),
    )(q, k)
